# baseline (speedup 1.0000x reference)
import jax
import jax.numpy as jnp
from jax.experimental import pallas as pl

_TAU_INV = 2.0


_NCHUNK = 4


def _body(l_ref, *refs):
    u_refs, o_ref = refs[:_NCHUNK], refs[_NCHUNK]
    K, Dc = u_refs[0].shape[1], u_refs[0].shape[2]
    ars = []
    s = None
    for i, u_ref in enumerate(u_refs):
        a = jnp.exp(l_ref[0, :, i * Dc:(i + 1) * Dc] * _TAU_INV)
        t = jnp.log(u_ref[0])
        ar = a / (t * t)
        ars.append(ar)
        p = jnp.sum(ar, axis=1, keepdims=True)
        s = p if s is None else s + p
    r = 1.0 / s
    for i, ar in enumerate(ars):
        o_ref[0, :, i * Dc:(i + 1) * Dc] = jnp.max(
            ar * r, axis=0, keepdims=True)


def kernel(logits, uniform):
    B, K, D = uniform.shape
    Dc = D // _NCHUNK
    u_specs = [
        pl.BlockSpec((1, K, Dc), lambda b, i=i: (b, 0, i))
        for i in range(_NCHUNK)
    ]
    out = pl.pallas_call(
        _body,
        grid=(B,),
        in_specs=[pl.BlockSpec((1, 1, D), lambda b: (b, 0, 0))] + u_specs,
        out_specs=pl.BlockSpec((1, 1, D), lambda b: (b, 0, 0)),
        out_shape=jax.ShapeDtypeStruct((B, 1, D), jnp.float32),
    )(logits.reshape(B, 1, D), *([uniform] * _NCHUNK))
    return out.reshape(B, D)
